# Initial kernel scaffold; baseline (speedup 1.0000x reference)
import jax
import jax.numpy as jnp
from jax.experimental import pallas as pl


def kernel(x, edge_index, W1, b1, W2, b2, W3, b3):
    raise NotImplementedError("write your pallas kernel here")


# speedup vs baseline: 11.3678x; 11.3678x over previous
import functools

import jax
import jax.numpy as jnp
from jax import lax
from jax.experimental import pallas as pl
from jax.experimental.pallas import tpu as pltpu
from jax.experimental.pallas import tpu_sc as plsc

NC = 2
NS = 16
NL = 16
NW = NC * NS

_mesh = plsc.VectorSubcoreMesh(core_axis_name="c", subcore_axis_name="s")


def _make_hist(CHH, NP, W):
    RPT = NP // NS

    @functools.partial(
        pl.kernel,
        mesh=_mesh,
        out_type=jax.ShapeDtypeStruct((NC, NP, W), jnp.float32),
        scratch_types=[
            pltpu.VMEM((CHH, 128), jnp.int32),
            pltpu.VMEM((128, W), jnp.float32),
            pltpu.VMEM((128, W), jnp.float32),
            pltpu.VMEM_SHARED((NP, W), jnp.float32),
        ],
    )
    def hist(dsth_hbm, ones_hbm, degp_hbm, idx_v, ones_v, buf_v, deg_sh):
        c = lax.axis_index("c")
        s = lax.axis_index("s")
        pltpu.sync_copy(dsth_hbm.at[c, s], idx_v)
        pltpu.sync_copy(ones_hbm.at[0], ones_v)
        pltpu.sync_copy(ones_hbm.at[1], buf_v)

        def _zero_sh(i, _):
            pltpu.sync_copy(buf_v, deg_sh.at[pl.ds(s * RPT + i * 128, 128)])
            return _

        lax.fori_loop(0, RPT // 128, _zero_sh, None)
        plsc.subcore_barrier()

        def _accum(j, _):
            pltpu.sync_copy(ones_v, deg_sh.at[idx_v.at[j]], add=True)
            return _

        lax.fori_loop(0, CHH, _accum, None)
        plsc.subcore_barrier()

        def _writeout(i, _):
            pltpu.sync_copy(deg_sh.at[pl.ds(s * RPT + i * 128, 128)], buf_v)
            pltpu.sync_copy(buf_v, degp_hbm.at[c, pl.ds(s * RPT + i * 128, 128)])
            return _

        lax.fori_loop(0, RPT // 128, _writeout, None)

    return hist


def _make_scatter(CH, NP, H):
    RPT = NP // NS
    ZB = 64

    @functools.partial(
        pl.kernel,
        mesh=_mesh,
        out_type=jax.ShapeDtypeStruct((NC, NP, H), jnp.float32),
        scratch_types=[
            pltpu.VMEM((CH, 128), jnp.int32),
            pltpu.VMEM((CH, 128), jnp.int32),
            pltpu.VMEM((128, H), jnp.float32),
            pltpu.VMEM((ZB, H), jnp.float32),
            pltpu.VMEM_SHARED((NP, H), jnp.float32),
            pltpu.SemaphoreType.DMA,
        ],
    )
    def scat(y_hbm, srcp_hbm, dstp_hbm, zp_hbm, src_v, dst_v, rows_v, zbuf_v,
             z_sh, sem):
        c = lax.axis_index("c")
        s = lax.axis_index("s")
        pltpu.sync_copy(srcp_hbm.at[c, s], src_v)
        pltpu.sync_copy(dstp_hbm.at[c, s], dst_v)

        def _zero_zbuf(i, _):
            for k in range(H // NL):
                zbuf_v[i, pl.ds(k * NL, NL)] = jnp.zeros((NL,), jnp.float32)
            return _

        lax.fori_loop(0, ZB, _zero_zbuf, None)

        def _zero_sh(i, _):
            pltpu.sync_copy(zbuf_v, z_sh.at[pl.ds(s * RPT + i * ZB, ZB)])
            return _

        lax.fori_loop(0, RPT // ZB, _zero_sh, None)
        plsc.subcore_barrier()

        def _edges(j, _):
            pltpu.async_copy(y_hbm.at[src_v.at[j]], rows_v, sem).wait()
            pltpu.sync_copy(rows_v, z_sh.at[dst_v.at[j]], add=True)
            return _

        lax.fori_loop(0, CH, _edges, None)
        plsc.subcore_barrier()

        def _writeout(i, _):
            pltpu.sync_copy(z_sh.at[pl.ds(s * RPT + i * 128, 128)], rows_v)
            pltpu.sync_copy(rows_v, zp_hbm.at[c, pl.ds(s * RPT + i * 128, 128)])
            return _

        lax.fori_loop(0, RPT // 128, _writeout, None)

    return scat


def _prep_body(degp_ref, x_ref, w_ref, y_ref, dinv_ref):
    deg = jnp.sum(degp_ref[...], axis=0)
    dinv = jnp.where(deg > 0, lax.rsqrt(deg), 0.0)
    y = jnp.dot(x_ref[...], w_ref[...], preferred_element_type=jnp.float32)
    y_ref[...] = y * dinv
    dinv_ref[...] = dinv


def _mid_body(zp_ref, y_ref, dinv_ref, b_ref, w_ref, yn_ref):
    z = zp_ref[0] + zp_ref[1] + y_ref[...]
    xn = jnp.maximum(z * dinv_ref[...] + b_ref[...], 0.0)
    yn = jnp.dot(xn, w_ref[...], preferred_element_type=jnp.float32)
    yn_ref[...] = yn * dinv_ref[...]


def _post_body(zp_ref, y_ref, dinv_ref, b_ref, out_ref):
    z = zp_ref[0] + zp_ref[1] + y_ref[...]
    out_ref[...] = z * dinv_ref[...] + b_ref[...]


def _tc_grid(NP, H, R):
    grid = NP // R
    row = pl.BlockSpec((R, H), lambda i: (i, 0))
    col = pl.BlockSpec((R, 1), lambda i: (i, 0))
    zsp = pl.BlockSpec((NC, R, H), lambda i: (0, i, 0))
    dsp = pl.BlockSpec((NC, R, 1), lambda i: (0, i, 0))
    wsp = pl.BlockSpec((H, H), lambda i: (0, 0))
    bsp = pl.BlockSpec((1, H), lambda i: (0, 0))
    return grid, row, col, zsp, dsp, wsp, bsp


def _make_tc(NP, H, R=1280):
    grid, row, col, zsp, dsp, wsp, bsp = _tc_grid(NP, H, R)
    f32 = jnp.float32
    prep = pl.pallas_call(
        _prep_body,
        grid=grid,
        in_specs=[dsp, row, wsp],
        out_specs=[row, col],
        out_shape=[
            jax.ShapeDtypeStruct((NP, H), f32),
            jax.ShapeDtypeStruct((NP, 1), f32),
        ],
    )
    mid = pl.pallas_call(
        _mid_body,
        grid=grid,
        in_specs=[zsp, row, col, bsp, wsp],
        out_specs=row,
        out_shape=jax.ShapeDtypeStruct((NP, H), f32),
    )
    post = pl.pallas_call(
        _post_body,
        grid=grid,
        in_specs=[zsp, row, col, bsp],
        out_specs=row,
        out_shape=jax.ShapeDtypeStruct((NP, H), f32),
    )
    return prep, mid, post


def kernel(x, edge_index, W1, b1, W2, b2, W3, b3):
    N, D = x.shape
    H = W1.shape[1]
    E = edge_index.shape[1]
    NP = ((N + 2560) // 2560) * 2560
    src = edge_index[0]
    dst = edge_index[1]
    EH = E + N
    CHH = ((EH + NW - 1) // NW + 127) // 128
    dsth = jnp.concatenate([dst, jnp.arange(N, dtype=jnp.int32)])
    dsth = jnp.pad(dsth, (0, NW * CHH * 128 - EH), constant_values=N)
    dsth = dsth.reshape(NC, NS, CHH, 128)
    CH = ((E + NW - 1) // NW + 127) // 128
    EP = NW * CH * 128
    srcp = jnp.pad(src, (0, EP - E)).reshape(NC, NS, CH, 128)
    dstp = jnp.pad(dst, (0, EP - E), constant_values=N).reshape(NC, NS, CH, 128)
    x_pad = jnp.pad(x, ((0, NP - N), (0, 0)))

    HW = 128
    hist = _make_hist(CHH, NP, HW)
    scat = _make_scatter(CH, NP, H)
    prep, mid, post = _make_tc(NP, H)

    ones_arr = jnp.stack([
        jnp.ones((128, HW), jnp.float32), jnp.zeros((128, HW), jnp.float32)
    ])
    degp = hist(dsth, ones_arr)[:, :, :1]
    y1, dinv = prep(degp, x_pad, W1)
    zp1 = scat(y1, srcp, dstp)
    y2 = mid(zp1, y1, dinv, b1.reshape(1, H), W2)
    zp2 = scat(y2, srcp, dstp)
    y3 = mid(zp2, y2, dinv, b2.reshape(1, H), W3)
    zp3 = scat(y3, srcp, dstp)
    out = post(zp3, y3, dinv, b3.reshape(1, H))
    return out[:N]
